# baseline (speedup 1.0000x reference)
import functools

import jax
import jax.numpy as jnp
from jax import lax
from jax.experimental import pallas as pl
from jax.experimental.pallas import tpu as pltpu
from jax.experimental.pallas import tpu_sc as plsc

E = 8
IN = 4096
HID = 4096
N = 8192
TM = 256
NP = N + E * TM
NT = NP // TM
TN = 1024
NN = HID // TN

NW = 32
ROWS_W = N // NW
IW = IN // 2

_mesh = plsc.VectorSubcoreMesh(core_axis_name="c", subcore_axis_name="s")


CH = 16
NCH = ROWS_W // CH
CH2 = 8
NCH2 = ROWS_W // CH2


@functools.partial(
    pl.kernel, mesh=_mesh,
    out_type=jax.ShapeDtypeStruct((NP, IW), jnp.int32),
    scratch_types=[
        pltpu.VMEM((ROWS_W,), jnp.int32),
        pltpu.VMEM((CH, IW), jnp.int32),
        pltpu.VMEM((CH, IW), jnp.int32),
        pltpu.SemaphoreType.DMA,
        pltpu.SemaphoreType.DMA,
    ],
)
def _dispatch(dst_hbm, x_hbm, xs_hbm, dstv, bufa, bufb, sema, semb):
    wid = lax.axis_index("s") * 2 + lax.axis_index("c")
    base = wid * ROWS_W
    pltpu.sync_copy(dst_hbm.at[pl.ds(base, ROWS_W)], dstv)
    bufs = (bufa, bufb)
    sems = (sema, semb)
    pltpu.async_copy(x_hbm.at[pl.ds(base, CH)], bufa, sema).wait()
    for ch in range(NCH):
        cur = bufs[ch % 2]
        ld = None
        if ch + 1 < NCH:
            ld = pltpu.async_copy(
                x_hbm.at[pl.ds(base + (ch + 1) * CH, CH)],
                bufs[(ch + 1) % 2], sems[(ch + 1) % 2])
        idx = dstv[pl.ds(ch * CH, CH)]
        pltpu.async_copy(cur, xs_hbm.at[idx], sems[ch % 2]).wait()
        if ld is not None:
            ld.wait()


@functools.partial(
    pl.kernel, mesh=_mesh,
    out_type=jax.ShapeDtypeStruct((N, HID), jnp.float32),
    scratch_types=[
        pltpu.VMEM((ROWS_W,), jnp.int32),
        pltpu.VMEM((CH2, HID), jnp.float32),
        pltpu.VMEM((CH2, HID), jnp.float32),
        pltpu.SemaphoreType.DMA,
        pltpu.SemaphoreType.DMA,
    ],
)
def _regather(dst_hbm, ys_hbm, out_hbm, dstv, bufa, bufb, sema, semb):
    wid = lax.axis_index("s") * 2 + lax.axis_index("c")
    base = wid * ROWS_W
    pltpu.sync_copy(dst_hbm.at[pl.ds(base, ROWS_W)], dstv)
    bufs = (bufa, bufb)
    sems = (sema, semb)
    pltpu.async_copy(ys_hbm.at[dstv.at[pl.ds(0, CH2)]], bufa, sema).wait()
    for ch in range(NCH2):
        cur = bufs[ch % 2]
        ld = None
        if ch + 1 < NCH2:
            ld = pltpu.async_copy(
                ys_hbm.at[dstv.at[pl.ds((ch + 1) * CH2, CH2)]],
                bufs[(ch + 1) % 2], sems[(ch + 1) % 2])
        pltpu.sync_copy(cur, out_hbm.at[pl.ds(base + ch * CH2, CH2)])
        if ld is not None:
            ld.wait()


def _mm_body(te_ref, x_ref, w_ref, b_ref, o_ref, wbf):
    m = pl.program_id(1)
    e = te_ref[m]
    prev = te_ref[jnp.maximum(m - 1, 0)]
    changed = jnp.logical_or(m == 0, e != prev)

    @pl.when(changed)
    def _():
        wbf[...] = w_ref[0].astype(jnp.bfloat16)

    acc = jnp.dot(x_ref[...], wbf[...], preferred_element_type=jnp.float32)
    o_ref[...] = acc + b_ref[0]


def _grouped_matmul(tile_expert, xs_bf, W, b):
    grid_spec = pltpu.PrefetchScalarGridSpec(
        num_scalar_prefetch=1,
        grid=(NN, NT),
        in_specs=[
            pl.BlockSpec((TM, IN), lambda n, m, te: (m, 0)),
            pl.BlockSpec((1, IN, TN), lambda n, m, te: (te[m], 0, n)),
            pl.BlockSpec((1, 1, TN), lambda n, m, te: (te[m], 0, n)),
        ],
        out_specs=pl.BlockSpec((TM, TN), lambda n, m, te: (m, n)),
        scratch_shapes=[pltpu.VMEM((IN, TN), jnp.bfloat16)],
    )
    return pl.pallas_call(
        _mm_body,
        grid_spec=grid_spec,
        out_shape=jax.ShapeDtypeStruct((NP, HID), jnp.float32),
        compiler_params=pltpu.CompilerParams(
            dimension_semantics=("arbitrary", "arbitrary"),
        ),
    )(tile_expert, xs_bf, W, b.reshape(E, 1, HID))


def kernel(X, d, W, b):
    counts = jnp.bincount(d, length=E)
    padded = (counts + TM - 1) & ~(TM - 1)
    ends = jnp.cumsum(padded)
    off = ends - padded
    starts = jnp.cumsum(counts) - counts
    perm = jnp.argsort(d, stable=True)
    es = d[perm]
    dst_sorted = off[es] + (jnp.arange(N, dtype=jnp.int32) - starts[es])
    dst = jnp.zeros((N,), jnp.int32).at[perm].set(dst_sorted.astype(jnp.int32))
    tile_expert = jnp.minimum(
        jnp.searchsorted(ends, jnp.arange(NT) * TM, side="right"), E - 1
    ).astype(jnp.int32)

    xb = X.astype(jnp.bfloat16)
    xi = jax.lax.bitcast_convert_type(xb.reshape(N, IW, 2), jnp.int32)
    xs_i = _dispatch(dst, xi)
    xs = jax.lax.bitcast_convert_type(xs_i, jnp.bfloat16).reshape(NP, IN)

    ys = _grouped_matmul(tile_expert, xs, W, b)

    return _regather(dst, ys)


# speedup vs baseline: 2.3384x; 2.3384x over previous
import functools

import jax
import jax.numpy as jnp
from jax import lax
from jax.experimental import pallas as pl
from jax.experimental.pallas import tpu as pltpu
from jax.experimental.pallas import tpu_sc as plsc

E = 8
IN = 4096
HID = 4096
N = 8192
TM = 256
NP = N + E * TM
NT = NP // TM
TN = 1024
NN = HID // TN

NW = 32
ROWS_W = N // NW
IW = IN // 2

_mesh = plsc.VectorSubcoreMesh(core_axis_name="c", subcore_axis_name="s")


CH = 16
NCH = ROWS_W // CH
CH2 = 8
NCH2 = ROWS_W // CH2


HIN = IN // 2
NSUB = 2 * NCH


@functools.partial(
    pl.kernel, mesh=_mesh,
    out_type=(
        jax.ShapeDtypeStruct((NP, HIN), jnp.float32),
        jax.ShapeDtypeStruct((NP, HIN), jnp.float32),
    ),
    scratch_types=[
        pltpu.VMEM((ROWS_W,), jnp.int32),
        pltpu.VMEM((CH, HIN), jnp.float32),
        pltpu.VMEM((CH, HIN), jnp.float32),
        pltpu.SemaphoreType.DMA,
        pltpu.SemaphoreType.DMA,
        pltpu.SemaphoreType.DMA,
        pltpu.SemaphoreType.DMA,
    ],
)
def _dispatch(dst_hbm, x_hbm, xsl_hbm, xsr_hbm, dstv, bufa, bufb,
              la, lb, sa, sb):
    wid = lax.axis_index("s") * 2 + lax.axis_index("c")
    base = wid * ROWS_W
    pltpu.sync_copy(dst_hbm.at[pl.ds(base, ROWS_W)], dstv)
    bufs = (bufa, bufb)
    lsems = (la, lb)
    ssems = (sa, sb)
    outs = (xsl_hbm, xsr_hbm)

    def src(i):
        return x_hbm.at[pl.ds(base + (i // 2) * CH, CH),
                        pl.ds((i % 2) * HIN, HIN)]

    ldh = [None, None]
    sth = [None, None]
    ldh[0] = pltpu.async_copy(src(0), bufa, la)
    for i in range(NSUB):
        bb = i % 2
        nb = (i + 1) % 2
        if i + 1 < NSUB:
            if sth[nb] is not None:
                sth[nb].wait()
            ldh[nb] = pltpu.async_copy(src(i + 1), bufs[nb], lsems[nb])
        ldh[bb].wait()
        idx = dstv[pl.ds((i // 2) * CH, CH)]
        sth[bb] = pltpu.async_copy(bufs[bb], outs[bb].at[idx], ssems[bb])
    sth[0].wait()
    sth[1].wait()


def _cvt_body(l_ref, r_ref, o_ref):
    o_ref[:, 0:HIN] = l_ref[...].astype(jnp.bfloat16)
    o_ref[:, HIN:IN] = r_ref[...].astype(jnp.bfloat16)


def _convert(xsl, xsr):
    CR = 512
    return pl.pallas_call(
        _cvt_body,
        grid=(NP // CR,),
        in_specs=[
            pl.BlockSpec((CR, HIN), lambda i: (i, 0)),
            pl.BlockSpec((CR, HIN), lambda i: (i, 0)),
        ],
        out_specs=pl.BlockSpec((CR, IN), lambda i: (i, 0)),
        out_shape=jax.ShapeDtypeStruct((NP, IN), jnp.bfloat16),
    )(xsl, xsr)


@functools.partial(
    pl.kernel, mesh=_mesh,
    out_type=jax.ShapeDtypeStruct((N, HID), jnp.float32),
    scratch_types=[
        pltpu.VMEM((ROWS_W,), jnp.int32),
        pltpu.VMEM((CH2, HID), jnp.float32),
        pltpu.VMEM((CH2, HID), jnp.float32),
        pltpu.SemaphoreType.DMA,
        pltpu.SemaphoreType.DMA,
    ],
)
def _regather(dst_hbm, ys_hbm, out_hbm, dstv, bufa, bufb, sema, semb):
    wid = lax.axis_index("s") * 2 + lax.axis_index("c")
    base = wid * ROWS_W
    pltpu.sync_copy(dst_hbm.at[pl.ds(base, ROWS_W)], dstv)
    bufs = (bufa, bufb)
    sems = (sema, semb)
    pltpu.async_copy(ys_hbm.at[dstv.at[pl.ds(0, CH2)]], bufa, sema).wait()
    for ch in range(NCH2):
        cur = bufs[ch % 2]
        ld = None
        if ch + 1 < NCH2:
            ld = pltpu.async_copy(
                ys_hbm.at[dstv.at[pl.ds((ch + 1) * CH2, CH2)]],
                bufs[(ch + 1) % 2], sems[(ch + 1) % 2])
        pltpu.sync_copy(cur, out_hbm.at[pl.ds(base + ch * CH2, CH2)])
        if ld is not None:
            ld.wait()


def _mm_body(te_ref, x_ref, w_ref, b_ref, o_ref, wbf):
    m = pl.program_id(1)
    e = te_ref[m]
    prev = te_ref[jnp.maximum(m - 1, 0)]
    changed = jnp.logical_or(m == 0, e != prev)

    @pl.when(changed)
    def _():
        wbf[...] = w_ref[0].astype(jnp.bfloat16)

    acc = jnp.dot(x_ref[...], wbf[...], preferred_element_type=jnp.float32)
    o_ref[...] = acc + b_ref[0]


def _grouped_matmul(tile_expert, xs_bf, W, b):
    grid_spec = pltpu.PrefetchScalarGridSpec(
        num_scalar_prefetch=1,
        grid=(NN, NT),
        in_specs=[
            pl.BlockSpec((TM, IN), lambda n, m, te: (m, 0)),
            pl.BlockSpec((1, IN, TN), lambda n, m, te: (te[m], 0, n)),
            pl.BlockSpec((1, 1, TN), lambda n, m, te: (te[m], 0, n)),
        ],
        out_specs=pl.BlockSpec((TM, TN), lambda n, m, te: (m, n)),
        scratch_shapes=[pltpu.VMEM((IN, TN), jnp.bfloat16)],
    )
    return pl.pallas_call(
        _mm_body,
        grid_spec=grid_spec,
        out_shape=jax.ShapeDtypeStruct((NP, HID), jnp.float32),
        compiler_params=pltpu.CompilerParams(
            dimension_semantics=("arbitrary", "arbitrary"),
        ),
    )(tile_expert, xs_bf, W, b.reshape(E, 1, HID))


def kernel(X, d, W, b):
    counts = jnp.bincount(d, length=E)
    padded = (counts + TM - 1) & ~(TM - 1)
    ends = jnp.cumsum(padded)
    off = ends - padded
    starts = jnp.cumsum(counts) - counts
    perm = jnp.argsort(d, stable=True)
    es = d[perm]
    dst_sorted = off[es] + (jnp.arange(N, dtype=jnp.int32) - starts[es])
    dst = jnp.zeros((N,), jnp.int32).at[perm].set(dst_sorted.astype(jnp.int32))
    tile_expert = jnp.minimum(
        jnp.searchsorted(ends, jnp.arange(NT) * TM, side="right"), E - 1
    ).astype(jnp.int32)

    xsl, xsr = _dispatch(dst, X)
    xs = _convert(xsl, xsr)

    ys = _grouped_matmul(tile_expert, xs, W, b)

    return _regather(dst, ys)


# speedup vs baseline: 2.4700x; 1.0563x over previous
import functools

import jax
import jax.numpy as jnp
from jax import lax
from jax.experimental import pallas as pl
from jax.experimental.pallas import tpu as pltpu
from jax.experimental.pallas import tpu_sc as plsc

E = 8
IN = 4096
HID = 4096
N = 8192
TM = 256
NP = N + E * TM
NT = NP // TM
TN = 1024
NN = HID // TN

NW = 32
ROWS_W = N // NW
IW = IN // 2

_mesh = plsc.VectorSubcoreMesh(core_axis_name="c", subcore_axis_name="s")


CH = 16
NCH = ROWS_W // CH
CH2 = 8
NCH2 = ROWS_W // CH2


HIN = IN // 2
NSUB = 2 * NCH


NTE = 48


@functools.partial(
    pl.kernel, mesh=_mesh,
    out_type=(
        jax.ShapeDtypeStruct((NP, HIN), jnp.float32),
        jax.ShapeDtypeStruct((NP, HIN), jnp.float32),
        jax.ShapeDtypeStruct((N,), jnp.int32),
        jax.ShapeDtypeStruct((NTE,), jnp.int32),
    ),
    scratch_types=[
        pltpu.VMEM((N,), jnp.int32),
        pltpu.VMEM((ROWS_W,), jnp.int32),
        pltpu.VMEM((NTE,), jnp.int32),
        pltpu.VMEM((16,), jnp.int32),
        pltpu.VMEM((16,), jnp.int32),
        pltpu.VMEM((CH, HIN), jnp.float32),
        pltpu.VMEM((CH, HIN), jnp.float32),
        pltpu.SemaphoreType.DMA,
        pltpu.SemaphoreType.DMA,
        pltpu.SemaphoreType.DMA,
        pltpu.SemaphoreType.DMA,
    ],
    compiler_params=pltpu.CompilerParams(needs_layout_passes=False),
)
def _dispatch(d_hbm, x_hbm, xsl_hbm, xsr_hbm, dst_hbm, te_hbm,
              dv, dstv, tev, tot_ref, pre_ref, bufa, bufb, la, lb, sa, sb):
    wid = lax.axis_index("s") * 2 + lax.axis_index("c")
    base = wid * ROWS_W
    pltpu.sync_copy(d_hbm, dv)

    wvreg = wid * (ROWS_W // 16)
    ones = jnp.ones((16,), jnp.int32)
    lane = lax.iota(jnp.int32, 16)

    sel = [(lane == e).astype(jnp.int32) for e in range(E)]

    def hist_step(i, carry):
        tot_v, pre_v = carry
        snap = jnp.broadcast_to(i == wvreg, (16,))
        pre_v = jnp.where(snap, tot_v, pre_v)
        v = dv[pl.ds(i * 16, 16)]
        for e in range(E):
            cnt = plsc.all_reduce_population_count(v == e)
            tot_v = tot_v + cnt * sel[e]
        return tot_v, pre_v

    z16 = jnp.zeros((16,), jnp.int32)
    tot_v, pre_v = lax.fori_loop(0, N // 16, hist_step, (z16, z16))

    pad_v = jnp.bitwise_and(tot_v + (TM - 1), -TM)
    off_v = plsc.cumsum(pad_v) - pad_v

    def _splat(vec, e):
        s = jnp.sum(vec * (lane == e).astype(jnp.int32))
        return jnp.broadcast_to(s, (16,))

    off = [_splat(off_v, e) for e in range(E)]
    pad = [_splat(pad_v, e) for e in range(E)]
    running = [off[e] + _splat(pre_v, e) for e in range(E)]

    for k in range(ROWS_W // 16):
        v = dv[pl.ds(base + k * 16, 16)]
        dstk = jnp.zeros((16,), jnp.int32)
        for e in range(E):
            m = v == e
            mi = m.astype(jnp.int32)
            excl = plsc.cumsum(mi) - mi
            dstk = jnp.where(m, running[e] + excl, dstk)
            running[e] = running[e] + plsc.all_reduce_population_count(m)
        dstv[pl.ds(k * 16, 16)] = dstk
    pltpu.sync_copy(dstv, dst_hbm.at[pl.ds(base, ROWS_W)])

    @pl.when(wid == 0)
    def _():
        end_all = off[E - 1] + pad[E - 1]
        for k in range(NTE // 16):
            rb = (lax.iota(jnp.int32, 16) + (k * 16)) * TM
            te = jnp.zeros((16,), jnp.int32)
            for e in range(E):
                inside = (rb >= off[e]) & (rb < off[e] + pad[e])
                te = jnp.where(inside, jnp.full((16,), e, jnp.int32), te)
            te = jnp.where(rb >= end_all, jnp.full((16,), E - 1, jnp.int32), te)
            tev[pl.ds(k * 16, 16)] = te
        pltpu.sync_copy(tev, te_hbm)

    bufs = (bufa, bufb)
    lsems = (la, lb)
    ssems = (sa, sb)
    outs = (xsl_hbm, xsr_hbm)

    def src(i):
        return x_hbm.at[pl.ds(base + (i // 2) * CH, CH),
                        pl.ds((i % 2) * HIN, HIN)]

    ldh = [None, None]
    sth = [None, None]
    ldh[0] = pltpu.async_copy(src(0), bufa, la)
    for i in range(NSUB):
        bb = i % 2
        nb = (i + 1) % 2
        if i + 1 < NSUB:
            if sth[nb] is not None:
                sth[nb].wait()
            ldh[nb] = pltpu.async_copy(src(i + 1), bufs[nb], lsems[nb])
        ldh[bb].wait()
        idx = dstv[pl.ds((i // 2) * CH, CH)]
        sth[bb] = pltpu.async_copy(bufs[bb], outs[bb].at[idx], ssems[bb])
    sth[0].wait()
    sth[1].wait()


def _cvt_body(l_ref, r_ref, o_ref):
    o_ref[:, 0:HIN] = l_ref[...].astype(jnp.bfloat16)
    o_ref[:, HIN:IN] = r_ref[...].astype(jnp.bfloat16)


def _convert(xsl, xsr):
    CR = 512
    return pl.pallas_call(
        _cvt_body,
        grid=(NP // CR,),
        in_specs=[
            pl.BlockSpec((CR, HIN), lambda i: (i, 0)),
            pl.BlockSpec((CR, HIN), lambda i: (i, 0)),
        ],
        out_specs=pl.BlockSpec((CR, IN), lambda i: (i, 0)),
        out_shape=jax.ShapeDtypeStruct((NP, IN), jnp.bfloat16),
    )(xsl, xsr)


@functools.partial(
    pl.kernel, mesh=_mesh,
    out_type=jax.ShapeDtypeStruct((N, HID), jnp.float32),
    scratch_types=[
        pltpu.VMEM((ROWS_W,), jnp.int32),
        pltpu.VMEM((CH2, HID), jnp.float32),
        pltpu.VMEM((CH2, HID), jnp.float32),
        pltpu.SemaphoreType.DMA,
        pltpu.SemaphoreType.DMA,
    ],
)
def _regather(dst_hbm, ys_hbm, out_hbm, dstv, bufa, bufb, sema, semb):
    wid = lax.axis_index("s") * 2 + lax.axis_index("c")
    base = wid * ROWS_W
    pltpu.sync_copy(dst_hbm.at[pl.ds(base, ROWS_W)], dstv)
    bufs = (bufa, bufb)
    sems = (sema, semb)
    pltpu.async_copy(ys_hbm.at[dstv.at[pl.ds(0, CH2)]], bufa, sema).wait()
    for ch in range(NCH2):
        cur = bufs[ch % 2]
        ld = None
        if ch + 1 < NCH2:
            ld = pltpu.async_copy(
                ys_hbm.at[dstv.at[pl.ds((ch + 1) * CH2, CH2)]],
                bufs[(ch + 1) % 2], sems[(ch + 1) % 2])
        pltpu.sync_copy(cur, out_hbm.at[pl.ds(base + ch * CH2, CH2)])
        if ld is not None:
            ld.wait()


def _mm_body(te_ref, x_ref, w_ref, b_ref, o_ref, wbf):
    m = pl.program_id(1)
    e = te_ref[m]
    prev = te_ref[jnp.maximum(m - 1, 0)]
    changed = jnp.logical_or(m == 0, e != prev)

    @pl.when(changed)
    def _():
        wbf[...] = w_ref[0].astype(jnp.bfloat16)

    acc = jnp.dot(x_ref[...], wbf[...], preferred_element_type=jnp.float32)
    o_ref[...] = acc + b_ref[0]


def _grouped_matmul(tile_expert, xs_bf, W, b):
    grid_spec = pltpu.PrefetchScalarGridSpec(
        num_scalar_prefetch=1,
        grid=(NN, NT),
        in_specs=[
            pl.BlockSpec((TM, IN), lambda n, m, te: (m, 0)),
            pl.BlockSpec((1, IN, TN), lambda n, m, te: (te[m], 0, n)),
            pl.BlockSpec((1, 1, TN), lambda n, m, te: (te[m], 0, n)),
        ],
        out_specs=pl.BlockSpec((TM, TN), lambda n, m, te: (m, n)),
        scratch_shapes=[pltpu.VMEM((IN, TN), jnp.bfloat16)],
    )
    return pl.pallas_call(
        _mm_body,
        grid_spec=grid_spec,
        out_shape=jax.ShapeDtypeStruct((NP, HID), jnp.float32),
        compiler_params=pltpu.CompilerParams(
            dimension_semantics=("arbitrary", "arbitrary"),
        ),
    )(tile_expert, xs_bf, W, b.reshape(E, 1, HID))


def kernel(X, d, W, b):
    xsl, xsr, dst, tile_expert = _dispatch(d, X)
    xs = _convert(xsl, xsr)

    ys = _grouped_matmul(tile_expert, xs, W, b)

    return _regather(dst, ys)


# speedup vs baseline: 2.6796x; 1.0848x over previous
import functools

import jax
import jax.numpy as jnp
from jax import lax
from jax.experimental import pallas as pl
from jax.experimental.pallas import tpu as pltpu
from jax.experimental.pallas import tpu_sc as plsc

E = 8
IN = 4096
HID = 4096
N = 8192
TM = 256
NP = N + E * TM
NT = NP // TM
TN = 1024
NN = HID // TN

NW = 32
ROWS_W = N // NW
IW = IN // 2

_mesh = plsc.VectorSubcoreMesh(core_axis_name="c", subcore_axis_name="s")


CH = 16
NCH = ROWS_W // CH
CH2 = 8
NCH2 = ROWS_W // CH2


HIN = IN // 2
NSUB = 2 * NCH


NTE = 48


@functools.partial(
    pl.kernel, mesh=_mesh,
    out_type=(
        jax.ShapeDtypeStruct((NP, HIN), jnp.float32),
        jax.ShapeDtypeStruct((NP, HIN), jnp.float32),
        jax.ShapeDtypeStruct((N,), jnp.int32),
        jax.ShapeDtypeStruct((NTE,), jnp.int32),
        jax.ShapeDtypeStruct((NTE,), jnp.int32),
        jax.ShapeDtypeStruct((NTE,), jnp.int32),
    ),
    scratch_types=[
        pltpu.VMEM((N,), jnp.int32),
        pltpu.VMEM((ROWS_W,), jnp.int32),
        pltpu.VMEM((NTE,), jnp.int32),
        pltpu.VMEM((NTE,), jnp.int32),
        pltpu.VMEM((NTE,), jnp.int32),
        pltpu.VMEM((16,), jnp.int32),
        pltpu.VMEM((16,), jnp.int32),
        pltpu.VMEM((CH, HIN), jnp.float32),
        pltpu.VMEM((CH, HIN), jnp.float32),
        pltpu.SemaphoreType.DMA,
        pltpu.SemaphoreType.DMA,
        pltpu.SemaphoreType.DMA,
        pltpu.SemaphoreType.DMA,
    ],
    compiler_params=pltpu.CompilerParams(needs_layout_passes=False),
)
def _dispatch(d_hbm, x_hbm, xsl_hbm, xsr_hbm, dst_hbm, te_hbm, nxe_hbm,
              par_hbm, dv, dstv, tev, nxev, parv, tot_ref, pre_ref,
              bufa, bufb, la, lb, sa, sb):
    wid = lax.axis_index("s") * 2 + lax.axis_index("c")
    base = wid * ROWS_W
    pltpu.sync_copy(d_hbm, dv)

    wvreg = wid * (ROWS_W // 16)
    ones = jnp.ones((16,), jnp.int32)
    lane = lax.iota(jnp.int32, 16)

    sel = [(lane == e).astype(jnp.int32) for e in range(E)]

    def hist_step(i, carry):
        tot_v, pre_v = carry
        snap = jnp.broadcast_to(i == wvreg, (16,))
        pre_v = jnp.where(snap, tot_v, pre_v)
        v = dv[pl.ds(i * 16, 16)]
        for e in range(E):
            cnt = plsc.all_reduce_population_count(v == e)
            tot_v = tot_v + cnt * sel[e]
        return tot_v, pre_v

    z16 = jnp.zeros((16,), jnp.int32)
    tot_v, pre_v = lax.fori_loop(0, N // 16, hist_step, (z16, z16))

    pad_v = jnp.bitwise_and(tot_v + (TM - 1), -TM)
    off_v = plsc.cumsum(pad_v) - pad_v

    def _splat(vec, e):
        s = jnp.sum(vec * (lane == e).astype(jnp.int32))
        return jnp.broadcast_to(s, (16,))

    off = [_splat(off_v, e) for e in range(E)]
    pad = [_splat(pad_v, e) for e in range(E)]
    running = [off[e] + _splat(pre_v, e) for e in range(E)]

    for k in range(ROWS_W // 16):
        v = dv[pl.ds(base + k * 16, 16)]
        dstk = jnp.zeros((16,), jnp.int32)
        for e in range(E):
            m = v == e
            mi = m.astype(jnp.int32)
            excl = plsc.cumsum(mi) - mi
            dstk = jnp.where(m, running[e] + excl, dstk)
            running[e] = running[e] + plsc.all_reduce_population_count(m)
        dstv[pl.ds(k * 16, 16)] = dstk
    pltpu.sync_copy(dstv, dst_hbm.at[pl.ds(base, ROWS_W)])

    @pl.when(wid == 0)
    def _():
        end_all = off[E - 1] + pad[E - 1]
        present = [pad[e] for e in range(E)]
        nxt_of = [None] * E
        nxt_of[E - 1] = jnp.full((16,), E - 1, jnp.int32)
        for e in range(E - 2, -1, -1):
            nxt_of[e] = jnp.where(present[e + 1] > 0,
                                  jnp.full((16,), e + 1, jnp.int32),
                                  nxt_of[e + 1])
        runidx = [None] * E
        acc_r = jnp.zeros((16,), jnp.int32)
        for e in range(E):
            runidx[e] = acc_r
            acc_r = acc_r + (present[e] > 0).astype(jnp.int32)
        n_present = acc_r
        for k in range(NTE // 16):
            rb = (lax.iota(jnp.int32, 16) + (k * 16)) * TM
            te = jnp.zeros((16,), jnp.int32)
            nx = jnp.full((16,), E - 1, jnp.int32)
            rp = jnp.bitwise_and(n_present, 1)
            for e in range(E):
                inside = (rb >= off[e]) & (rb < off[e] + pad[e])
                te = jnp.where(inside, jnp.full((16,), e, jnp.int32), te)
                nx = jnp.where(inside, nxt_of[e], nx)
                rp = jnp.where(inside, jnp.bitwise_and(runidx[e], 1), rp)
            beyond = rb >= end_all
            te = jnp.where(beyond, jnp.full((16,), E - 1, jnp.int32), te)
            nx = jnp.where(beyond, jnp.full((16,), E - 1, jnp.int32), nx)
            tev[pl.ds(k * 16, 16)] = te
            nxev[pl.ds(k * 16, 16)] = nx
            parv[pl.ds(k * 16, 16)] = rp
        pltpu.sync_copy(tev, te_hbm)
        pltpu.sync_copy(nxev, nxe_hbm)
        pltpu.sync_copy(parv, par_hbm)

    bufs = (bufa, bufb)
    lsems = (la, lb)
    ssems = (sa, sb)
    outs = (xsl_hbm, xsr_hbm)

    def src(i):
        return x_hbm.at[pl.ds(base + (i // 2) * CH, CH),
                        pl.ds((i % 2) * HIN, HIN)]

    ldh = [None, None]
    sth = [None, None]
    ldh[0] = pltpu.async_copy(src(0), bufa, la)
    for i in range(NSUB):
        bb = i % 2
        nb = (i + 1) % 2
        if i + 1 < NSUB:
            if sth[nb] is not None:
                sth[nb].wait()
            ldh[nb] = pltpu.async_copy(src(i + 1), bufs[nb], lsems[nb])
        ldh[bb].wait()
        idx = dstv[pl.ds((i // 2) * CH, CH)]
        sth[bb] = pltpu.async_copy(bufs[bb], outs[bb].at[idx], ssems[bb])
    sth[0].wait()
    sth[1].wait()


def _cvt_body(l_ref, r_ref, o_ref):
    o_ref[:, 0:HIN] = l_ref[...].astype(jnp.bfloat16)
    o_ref[:, HIN:IN] = r_ref[...].astype(jnp.bfloat16)


def _convert(xsl, xsr):
    CR = 512
    return pl.pallas_call(
        _cvt_body,
        grid=(NP // CR,),
        in_specs=[
            pl.BlockSpec((CR, HIN), lambda i: (i, 0)),
            pl.BlockSpec((CR, HIN), lambda i: (i, 0)),
        ],
        out_specs=pl.BlockSpec((CR, IN), lambda i: (i, 0)),
        out_shape=jax.ShapeDtypeStruct((NP, IN), jnp.bfloat16),
    )(xsl, xsr)


@functools.partial(
    pl.kernel, mesh=_mesh,
    out_type=jax.ShapeDtypeStruct((N, HID), jnp.float32),
    scratch_types=[
        pltpu.VMEM((ROWS_W,), jnp.int32),
        pltpu.VMEM((CH2, HID), jnp.float32),
        pltpu.VMEM((CH2, HID), jnp.float32),
        pltpu.SemaphoreType.DMA,
        pltpu.SemaphoreType.DMA,
    ],
)
def _regather(dst_hbm, ys_hbm, out_hbm, dstv, bufa, bufb, sema, semb):
    wid = lax.axis_index("s") * 2 + lax.axis_index("c")
    base = wid * ROWS_W
    pltpu.sync_copy(dst_hbm.at[pl.ds(base, ROWS_W)], dstv)
    bufs = (bufa, bufb)
    sems = (sema, semb)
    pltpu.async_copy(ys_hbm.at[dstv.at[pl.ds(0, CH2)]], bufa, sema).wait()
    for ch in range(NCH2):
        cur = bufs[ch % 2]
        ld = None
        if ch + 1 < NCH2:
            ld = pltpu.async_copy(
                ys_hbm.at[dstv.at[pl.ds((ch + 1) * CH2, CH2)]],
                bufs[(ch + 1) % 2], sems[(ch + 1) % 2])
        pltpu.sync_copy(cur, out_hbm.at[pl.ds(base + ch * CH2, CH2)])
        if ld is not None:
            ld.wait()


def _mm_body(te_ref, nxe_ref, par_ref, x_ref, w_hbm, b_ref, o_ref,
             wraw, wbf, sem):
    n = pl.program_id(0)
    m = pl.program_id(1)
    e = te_ref[m]
    p = par_ref[m]
    nxe = nxe_ref[m]
    prev = te_ref[jnp.maximum(m - 1, 0)]
    run_start = jnp.logical_or(m == 0, e != prev)

    def wsrc(ei):
        return w_hbm.at[ei, :, pl.ds(n * TN, TN)]

    @pl.when(run_start)
    def _():
        @pl.when(m == 0)
        def _():
            cp = pltpu.make_async_copy(wsrc(e), wraw.at[p], sem.at[p])
            cp.start()
            cp.wait()

        @pl.when(m > 0)
        def _():
            pltpu.make_async_copy(wsrc(e), wraw.at[p], sem.at[p]).wait()

        @pl.when(e != nxe)
        def _():
            pltpu.make_async_copy(
                wsrc(nxe), wraw.at[1 - p], sem.at[1 - p]).start()

        wbf[...] = wraw[p].astype(jnp.bfloat16)

    acc = jnp.dot(x_ref[...], wbf[...], preferred_element_type=jnp.float32)
    o_ref[...] = acc + b_ref[0]


def _grouped_matmul(tile_expert, nxe, par, xs_bf, W, b):
    grid_spec = pltpu.PrefetchScalarGridSpec(
        num_scalar_prefetch=3,
        grid=(NN, NT),
        in_specs=[
            pl.BlockSpec((TM, IN), lambda n, m, te, nx, pr: (m, 0)),
            pl.BlockSpec(memory_space=pltpu.MemorySpace.HBM),
            pl.BlockSpec((1, 1, TN), lambda n, m, te, nx, pr: (te[m], 0, n)),
        ],
        out_specs=pl.BlockSpec((TM, TN), lambda n, m, te, nx, pr: (m, n)),
        scratch_shapes=[
            pltpu.VMEM((2, IN, TN), jnp.float32),
            pltpu.VMEM((IN, TN), jnp.bfloat16),
            pltpu.SemaphoreType.DMA((2,)),
        ],
    )
    return pl.pallas_call(
        _mm_body,
        grid_spec=grid_spec,
        out_shape=jax.ShapeDtypeStruct((NP, HID), jnp.float32),
        compiler_params=pltpu.CompilerParams(
            dimension_semantics=("arbitrary", "arbitrary"),
        ),
    )(tile_expert, nxe, par, xs_bf, W, b.reshape(E, 1, HID))


def kernel(X, d, W, b):
    xsl, xsr, dst, tile_expert, nxe, par = _dispatch(d, X)
    xs = _convert(xsl, xsr)

    ys = _grouped_matmul(tile_expert, nxe, par, xs, W, b)

    return _regather(dst, ys)
